# baseline (speedup 1.0000x reference)
import functools

import jax
import jax.numpy as jnp
from jax import lax
from jax.experimental import pallas as pl
from jax.experimental.pallas import tpu as pltpu
from jax.experimental.pallas import tpu_sc as plsc

N_FACTORS = 32
N_ROWS = 1000000
BATCH = 16384
NC = 2
NS = 16
NW = NC * NS
BPW = BATCH // NW
CHUNK = 128
NCH = BPW // CHUNK
LANES = 16

BLK = 1024


def _transpose_body(i_ref, o_ref):
    o_ref[...] = i_ref[...].T


_transpose_tc = pl.pallas_call(
    _transpose_body,
    grid=((N_ROWS + BLK - 1) // BLK,),
    in_specs=[pl.BlockSpec((N_FACTORS, BLK), lambda j: (0, j))],
    out_specs=pl.BlockSpec((BLK, N_FACTORS), lambda j: (j, 0)),
    out_shape=jax.ShapeDtypeStruct((N_ROWS, N_FACTORS), jnp.float32),
)


def _mf_body(user_r, item_r, uf_r, if_r, out_r,
             uidx, iidx, urows, irows, outv, sem):
    wid = lax.axis_index("s") * NC + lax.axis_index("c")

    pltpu.sync_copy(user_r.at[wid], uidx)
    pltpu.sync_copy(item_r.at[wid], iidx)

    lane = lax.iota(jnp.int32, LANES)

    for j in range(NCH):
        def issue(p0, c):
            uv = uidx[j, pl.ds(p0 * LANES, LANES)]
            iv = iidx[j, pl.ds(p0 * LANES, LANES)]
            for q in range(LANES):
                p = p0 * LANES + q
                pltpu.async_copy(uf_r.at[pl.ds(uv[q], 1)],
                                 urows.at[pl.ds(p, 1)], sem)
                pltpu.async_copy(if_r.at[pl.ds(iv[q], 1)],
                                 irows.at[pl.ds(p, 1)], sem)
            return c

        lax.fori_loop(0, CHUNK // LANES, issue, 0)

        pltpu.make_async_copy(uf_r.at[pl.ds(0, CHUNK)], urows, sem).wait()
        pltpu.make_async_copy(if_r.at[pl.ds(0, CHUNK)], irows, sem).wait()

        def group(gg, c):
            o = gg * LANES
            acc = jnp.zeros((LANES,), jnp.float32)
            for r in range(LANES):
                row = o + r
                s0 = urows[row, pl.ds(0, LANES)] * irows[row, pl.ds(0, LANES)]
                s1 = urows[row, pl.ds(LANES, LANES)] * irows[row, pl.ds(LANES, LANES)]
                tot = jnp.sum(s0 + s1)
                acc = jnp.where(lane == r, tot, acc)
            outv[pl.ds(j * CHUNK + o, LANES)] = acc
            return c

        lax.fori_loop(0, CHUNK // LANES, group, 0)

    pltpu.sync_copy(outv, out_r.at[wid])


_mf = functools.partial(
    pl.kernel,
    mesh=plsc.VectorSubcoreMesh(core_axis_name="c", subcore_axis_name="s"),
    out_type=jax.ShapeDtypeStruct((NW, BPW), jnp.float32),
    scratch_types=[
        pltpu.VMEM((NCH, CHUNK), jnp.int32),
        pltpu.VMEM((NCH, CHUNK), jnp.int32),
        pltpu.VMEM((CHUNK, N_FACTORS), jnp.float32),
        pltpu.VMEM((CHUNK, N_FACTORS), jnp.float32),
        pltpu.VMEM((BPW,), jnp.float32),
        pltpu.SemaphoreType.DMA,
    ],
    compiler_params=pltpu.CompilerParams(needs_layout_passes=False),
)(_mf_body)


def kernel(user, item, user_factors, item_factors):
    u = user.astype(jnp.int32).reshape(NW, NCH, CHUNK)
    i = item.astype(jnp.int32).reshape(NW, NCH, CHUNK)
    uf_rm = _transpose_tc(user_factors.T)
    if_rm = _transpose_tc(item_factors.T)
    out = _mf(u, i, uf_rm, if_rm)
    return out.reshape(BATCH)


# speedup vs baseline: 1.3821x; 1.3821x over previous
import functools

import jax
import jax.numpy as jnp
from jax import lax
from jax.experimental import pallas as pl
from jax.experimental.pallas import tpu as pltpu
from jax.experimental.pallas import tpu_sc as plsc

N_FACTORS = 32
N_ROWS = 1000000
SUP = 64
RPS = SUP // N_FACTORS
BATCH = 16384
NC = 2
NS = 16
NW = NC * NS
BPW = BATCH // NW
CHUNK = 128
NCH = BPW // CHUNK
LANES = 16


def _mf_body(user_r, item_r, uf_r, if_r, out_r,
             uidx, iidx, urows, irows, outv, sem):
    wid = lax.axis_index("s") * NC + lax.axis_index("c")

    pltpu.sync_copy(user_r.at[wid], uidx)
    pltpu.sync_copy(item_r.at[wid], iidx)

    lane = lax.iota(jnp.int32, LANES)

    for j in range(NCH):
        def issue(p0, c):
            uv = lax.shift_right_logical(uidx[j, pl.ds(p0 * LANES, LANES)], 1)
            iv = lax.shift_right_logical(iidx[j, pl.ds(p0 * LANES, LANES)], 1)
            for q in range(LANES):
                p = p0 * LANES + q
                pltpu.async_copy(uf_r.at[pl.ds(uv[q], 1)],
                                 urows.at[pl.ds(p, 1)], sem)
                pltpu.async_copy(if_r.at[pl.ds(iv[q], 1)],
                                 irows.at[pl.ds(p, 1)], sem)
            return c

        lax.fori_loop(0, CHUNK // LANES, issue, 0)

        pltpu.make_async_copy(uf_r.at[pl.ds(0, CHUNK)], urows, sem).wait()
        pltpu.make_async_copy(if_r.at[pl.ds(0, CHUNK)], irows, sem).wait()

        def group(gg, c):
            o = gg * LANES
            co_u = (uidx[j, pl.ds(o, LANES)] & (RPS - 1)) * N_FACTORS
            co_i = (iidx[j, pl.ds(o, LANES)] & (RPS - 1)) * N_FACTORS
            acc = jnp.zeros((LANES,), jnp.float32)
            for r in range(LANES):
                cou = co_u[r]
                coi = co_i[r]
                row = o + r
                s0 = urows[row, pl.ds(cou, LANES)] * irows[row, pl.ds(coi, LANES)]
                s1 = urows[row, pl.ds(cou + LANES, LANES)] * irows[row, pl.ds(coi + LANES, LANES)]
                tot = jnp.sum(s0 + s1)
                acc = jnp.where(lane == r, tot, acc)
            outv[pl.ds(j * CHUNK + o, LANES)] = acc
            return c

        lax.fori_loop(0, CHUNK // LANES, group, 0)

    pltpu.sync_copy(outv, out_r.at[wid])


_mf = functools.partial(
    pl.kernel,
    mesh=plsc.VectorSubcoreMesh(core_axis_name="c", subcore_axis_name="s"),
    out_type=jax.ShapeDtypeStruct((NW, BPW), jnp.float32),
    scratch_types=[
        pltpu.VMEM((NCH, CHUNK), jnp.int32),
        pltpu.VMEM((NCH, CHUNK), jnp.int32),
        pltpu.VMEM((CHUNK, SUP), jnp.float32),
        pltpu.VMEM((CHUNK, SUP), jnp.float32),
        pltpu.VMEM((BPW,), jnp.float32),
        pltpu.SemaphoreType.DMA,
    ],
    compiler_params=pltpu.CompilerParams(needs_layout_passes=False),
)(_mf_body)


def kernel(user, item, user_factors, item_factors):
    u = user.astype(jnp.int32).reshape(NW, NCH, CHUNK)
    i = item.astype(jnp.int32).reshape(NW, NCH, CHUNK)
    uf = user_factors.reshape(N_ROWS // RPS, SUP)
    itf = item_factors.reshape(N_ROWS // RPS, SUP)
    out = _mf(u, i, uf, itf)
    return out.reshape(BATCH)


# speedup vs baseline: 2.3259x; 1.6828x over previous
import functools

import jax
import jax.numpy as jnp
from jax import lax
from jax.experimental import pallas as pl
from jax.experimental.pallas import tpu as pltpu
from jax.experimental.pallas import tpu_sc as plsc

N_FACTORS = 32
BATCH = 16384
NC = 2
NS = 16
NW = NC * NS
BPW = BATCH // NW
CHUNK = 128
NCH = BPW // CHUNK
LANES = 16


def _mf_body(user_r, item_r, uf_r, if_r, out_r,
             uidx, iidx, urows, irows, outv, sem):
    wid = lax.axis_index("s") * NC + lax.axis_index("c")

    pltpu.sync_copy(user_r.at[wid], uidx)
    pltpu.sync_copy(item_r.at[wid], iidx)

    lane = lax.iota(jnp.int32, LANES)

    for j in range(NCH):
        def issue(p0, c):
            uv = uidx[j, pl.ds(p0 * LANES, LANES)]
            iv = iidx[j, pl.ds(p0 * LANES, LANES)]
            for q in range(LANES):
                p = p0 * LANES + q
                pltpu.async_copy(uf_r.at[pl.ds(uv[q], 1)],
                                 urows.at[pl.ds(p, 1)], sem)
                pltpu.async_copy(if_r.at[pl.ds(iv[q], 1)],
                                 irows.at[pl.ds(p, 1)], sem)
            return c

        lax.fori_loop(0, CHUNK // LANES, issue, 0)

        pltpu.make_async_copy(uf_r.at[pl.ds(0, CHUNK)], urows, sem).wait()
        pltpu.make_async_copy(if_r.at[pl.ds(0, CHUNK)], irows, sem).wait()

        def group(gg, c):
            o = gg * LANES
            acc = jnp.zeros((LANES,), jnp.float32)
            for r in range(LANES):
                row = o + r
                s0 = urows[row, pl.ds(0, LANES)] * irows[row, pl.ds(0, LANES)]
                s1 = urows[row, pl.ds(LANES, LANES)] * irows[row, pl.ds(LANES, LANES)]
                tot = jnp.sum(s0 + s1)
                acc = jnp.where(lane == r, tot, acc)
            outv[pl.ds(j * CHUNK + o, LANES)] = acc
            return c

        lax.fori_loop(0, CHUNK // LANES, group, 0)

    pltpu.sync_copy(outv, out_r.at[wid])


_mf = functools.partial(
    pl.kernel,
    mesh=plsc.VectorSubcoreMesh(core_axis_name="c", subcore_axis_name="s"),
    out_type=jax.ShapeDtypeStruct((NW, BPW), jnp.float32),
    scratch_types=[
        pltpu.VMEM((NCH, CHUNK), jnp.int32),
        pltpu.VMEM((NCH, CHUNK), jnp.int32),
        pltpu.VMEM((CHUNK, N_FACTORS), jnp.float32),
        pltpu.VMEM((CHUNK, N_FACTORS), jnp.float32),
        pltpu.VMEM((BPW,), jnp.float32),
        pltpu.SemaphoreType.DMA,
    ],
    compiler_params=pltpu.CompilerParams(needs_layout_passes=False),
)(_mf_body)


def kernel(user, item, user_factors, item_factors):
    u = user.astype(jnp.int32).reshape(NW, NCH, CHUNK)
    i = item.astype(jnp.int32).reshape(NW, NCH, CHUNK)
    out = _mf(u, i, user_factors, item_factors)
    return out.reshape(BATCH)
